# Initial kernel scaffold; baseline (speedup 1.0000x reference)
import jax
import jax.numpy as jnp
from jax import lax
from jax.experimental import pallas as pl
from jax.experimental.pallas import tpu as pltpu
from jax.experimental.pallas import tpu_sc as plsc

N_NODE = 100000
D = 128
B_POS = 16384
B_NEG = 81920

NC = 2
NS = 16
NW = NC * NS
LANES = 16
CHUNK = 128

POS_BLKS = B_POS // CHUNK
NEG_BLKS = B_NEG // CHUNK
POS_PER_W = POS_BLKS // NW
NEG_PER_W = NEG_BLKS // NW
DSUB = D // LANES


def _sc_body(pf, pt, nf, nt, e1, e2, e2c, o1, o2, on,
             ia, ib, ra, rb, rc, rd, p1, p2, sem):
    wid = lax.axis_index("s") * NC + lax.axis_index("c")

    def dots(src_a, src_b, part):
        def row(r, c):
            acc = src_a[r, pl.ds(0, LANES)] * src_b[r, pl.ds(0, LANES)]
            for b in range(1, DSUB):
                acc = acc + (src_a[r, pl.ds(LANES * b, LANES)]
                             * src_b[r, pl.ds(LANES * b, LANES)])
            part[r, :] = acc
            return c
        lax.fori_loop(0, CHUNK, row, 0)

    def pos_chunk(c, carry):
        blk = wid * POS_PER_W + c
        pltpu.sync_copy(pf.at[blk], ia)
        pltpu.sync_copy(pt.at[blk], ib)
        cps = [pltpu.async_copy(e1.at[ia], ra, sem),
               pltpu.async_copy(e1.at[ib], rb, sem),
               pltpu.async_copy(e2.at[ia], rc, sem),
               pltpu.async_copy(e2c.at[ib], rd, sem)]
        for cp in cps:
            cp.wait()
        dots(ra, rb, p1)
        dots(rc, rd, p2)
        pltpu.sync_copy(p1, o1.at[blk])
        pltpu.sync_copy(p2, o2.at[blk])
        return carry

    lax.fori_loop(0, POS_PER_W, pos_chunk, 0)

    def neg_chunk(c, carry):
        blk = wid * NEG_PER_W + c
        pltpu.sync_copy(nf.at[blk], ia)
        pltpu.sync_copy(nt.at[blk], ib)
        cp1 = pltpu.async_copy(e2.at[ia], ra, sem)
        cp2 = pltpu.async_copy(e2c.at[ib], rb, sem)
        cp1.wait()
        cp2.wait()
        dots(ra, rb, p1)
        pltpu.sync_copy(p1, on.at[blk])
        return carry

    lax.fori_loop(0, NEG_PER_W, neg_chunk, 0)


_sc_dots = pl.kernel(
    _sc_body,
    out_type=(
        jax.ShapeDtypeStruct((POS_BLKS, CHUNK, LANES), jnp.float32),
        jax.ShapeDtypeStruct((POS_BLKS, CHUNK, LANES), jnp.float32),
        jax.ShapeDtypeStruct((NEG_BLKS, CHUNK, LANES), jnp.float32),
    ),
    mesh=plsc.VectorSubcoreMesh(core_axis_name="c", subcore_axis_name="s"),
    scratch_types=(
        pltpu.VMEM((CHUNK,), jnp.int32),
        pltpu.VMEM((CHUNK,), jnp.int32),
        pltpu.VMEM((CHUNK, D), jnp.float32),
        pltpu.VMEM((CHUNK, D), jnp.float32),
        pltpu.VMEM((CHUNK, D), jnp.float32),
        pltpu.VMEM((CHUNK, D), jnp.float32),
        pltpu.VMEM((CHUNK, LANES), jnp.float32),
        pltpu.VMEM((CHUNK, LANES), jnp.float32),
        pltpu.SemaphoreType.DMA,
    ),
)


def _log_sigmoid(x):
    return jnp.minimum(x, 0.0) - jnp.log(1.0 + jnp.exp(-jnp.abs(x)))


def _reduce_body(o1, o2, on, w, first_ref, second_ref):
    s1 = jnp.sum(o1[...], axis=-1)
    s2 = jnp.sum(o2[...], axis=-1)
    sn = jnp.sum(on[...], axis=-1)
    first = -jnp.sum(w[...] * _log_sigmoid(s1))
    pos_loss = jnp.sum(_log_sigmoid(s2))
    neg_loss = jnp.sum(_log_sigmoid(-sn))
    first_ref[0, 0] = first
    second_ref[0, 0] = -(pos_loss + neg_loss)


_reduce = pl.pallas_call(
    _reduce_body,
    out_shape=(
        jax.ShapeDtypeStruct((1, 1), jnp.float32),
        jax.ShapeDtypeStruct((1, 1), jnp.float32),
    ),
)


def kernel(pos, pos_w, neg, embed_1, embed_2, embed_2_context):
    pf = pos[:, 0].reshape(POS_BLKS, CHUNK)
    pt = pos[:, 1].reshape(POS_BLKS, CHUNK)
    nf = neg[:, 0].reshape(NEG_BLKS, CHUNK)
    nt = neg[:, 1].reshape(NEG_BLKS, CHUNK)

    o1, o2, on = _sc_dots(pf, pt, nf, nt, embed_1, embed_2, embed_2_context)

    first, second = _reduce(o1, o2, on, pos_w.reshape(POS_BLKS, CHUNK))
    return first[0, 0], second[0, 0]


# speedup vs baseline: 1.0828x; 1.0828x over previous
import jax
import jax.numpy as jnp
from jax import lax
from jax.experimental import pallas as pl
from jax.experimental.pallas import tpu as pltpu
from jax.experimental.pallas import tpu_sc as plsc

N_NODE = 100000
D = 128
B_POS = 16384
B_NEG = 81920

NC = 2
NS = 16
NW = NC * NS
LANES = 16
CHUNK = 128

POS_BLKS = B_POS // CHUNK
NEG_BLKS = B_NEG // CHUNK
POS_PER_W = POS_BLKS // NW
NEG_PER_W = NEG_BLKS // NW
DSUB = D // LANES


def _sc_body(pf, pt, nf, nt, e1, e2, e2c, o1, o2, on,
             ia, ib, ra, rb, rc, rd, p1, p2, sem):
    wid = lax.axis_index("s") * NC + lax.axis_index("c")

    def dots(src_a, src_b, part):
        def row(r, c):
            acc = src_a[r, pl.ds(0, LANES)] * src_b[r, pl.ds(0, LANES)]
            for b in range(1, DSUB):
                acc = acc + (src_a[r, pl.ds(LANES * b, LANES)]
                             * src_b[r, pl.ds(LANES * b, LANES)])
            part[r, :] = acc
            return c
        lax.fori_loop(0, CHUNK, row, 0)

    def pos_chunk(c, carry):
        blk = wid * POS_PER_W + c
        pltpu.sync_copy(pf.at[blk], ia)
        pltpu.sync_copy(pt.at[blk], ib)
        cps = [pltpu.async_copy(e1.at[ia], ra, sem),
               pltpu.async_copy(e1.at[ib], rb, sem),
               pltpu.async_copy(e2.at[ia], rc, sem),
               pltpu.async_copy(e2c.at[ib], rd, sem)]
        for cp in cps:
            cp.wait()
        dots(ra, rb, p1)
        dots(rc, rd, p2)
        pltpu.sync_copy(p1, o1.at[blk])
        pltpu.sync_copy(p2, o2.at[blk])
        return carry

    lax.fori_loop(0, POS_PER_W, pos_chunk, 0)

    def neg_chunk(c, carry):
        blk = wid * NEG_PER_W + c
        pltpu.sync_copy(nf.at[blk], ia)
        pltpu.sync_copy(nt.at[blk], ib)
        cp1 = pltpu.async_copy(e2.at[ia], ra, sem)
        cp2 = pltpu.async_copy(e2c.at[ib], rb, sem)
        cp1.wait()
        cp2.wait()
        dots(ra, rb, p1)
        pltpu.sync_copy(p1, on.at[blk])
        return carry

    lax.fori_loop(0, NEG_PER_W, neg_chunk, 0)


_sc_dots = pl.kernel(
    _sc_body,
    out_type=(
        jax.ShapeDtypeStruct((POS_BLKS, CHUNK, LANES), jnp.float32),
        jax.ShapeDtypeStruct((POS_BLKS, CHUNK, LANES), jnp.float32),
        jax.ShapeDtypeStruct((NEG_BLKS, CHUNK, LANES), jnp.float32),
    ),
    mesh=plsc.VectorSubcoreMesh(core_axis_name="c", subcore_axis_name="s"),
    scratch_types=(
        pltpu.VMEM((CHUNK,), jnp.int32),
        pltpu.VMEM((CHUNK,), jnp.int32),
        pltpu.VMEM((CHUNK, D), jnp.float32),
        pltpu.VMEM((CHUNK, D), jnp.float32),
        pltpu.VMEM((CHUNK, D), jnp.float32),
        pltpu.VMEM((CHUNK, D), jnp.float32),
        pltpu.VMEM((CHUNK, LANES), jnp.float32),
        pltpu.VMEM((CHUNK, LANES), jnp.float32),
        pltpu.SemaphoreType.DMA,
    ),
)


def _log_sigmoid(x):
    return jnp.minimum(x, 0.0) - jnp.log(1.0 + jnp.exp(-jnp.abs(x)))


def _reduce_body(o1, o2, on, w, first_ref, second_ref):
    s1 = jnp.sum(o1[...], axis=-1)
    s2 = jnp.sum(o2[...], axis=-1)
    sn = jnp.sum(on[...], axis=-1)
    first = -jnp.sum(w[...] * _log_sigmoid(s1))
    pos_loss = jnp.sum(_log_sigmoid(s2))
    neg_loss = jnp.sum(_log_sigmoid(-sn))
    first_ref[0, 0] = first
    second_ref[0, 0] = -(pos_loss + neg_loss)


_reduce = pl.pallas_call(
    _reduce_body,
    out_shape=(
        jax.ShapeDtypeStruct((1, 1), jnp.float32),
        jax.ShapeDtypeStruct((1, 1), jnp.float32),
    ),
    out_specs=(
        pl.BlockSpec(memory_space=pltpu.SMEM),
        pl.BlockSpec(memory_space=pltpu.SMEM),
    ),
)


def kernel(pos, pos_w, neg, embed_1, embed_2, embed_2_context):
    pf = pos[:, 0].reshape(POS_BLKS, CHUNK)
    pt = pos[:, 1].reshape(POS_BLKS, CHUNK)
    nf = neg[:, 0].reshape(NEG_BLKS, CHUNK)
    nt = neg[:, 1].reshape(NEG_BLKS, CHUNK)

    o1, o2, on = _sc_dots(pf, pt, nf, nt, embed_1, embed_2, embed_2_context)

    first, second = _reduce(o1, o2, on, pos_w.reshape(POS_BLKS, CHUNK))
    return first[0, 0], second[0, 0]


# speedup vs baseline: 1.4813x; 1.3680x over previous
import jax
import jax.numpy as jnp
from jax import lax
from jax.experimental import pallas as pl
from jax.experimental.pallas import tpu as pltpu
from jax.experimental.pallas import tpu_sc as plsc

N_NODE = 100000
D = 128
B_POS = 16384
B_NEG = 81920

NC = 2
NS = 16
NW = NC * NS
LANES = 16
CHUNK = 128

POS_BLKS = B_POS // CHUNK
NEG_BLKS = B_NEG // CHUNK
POS_PER_W = POS_BLKS // NW
NEG_PER_W = NEG_BLKS // NW
DSUB = D // LANES


def _sc_body(pf, pt, nf, nt, e1, e2, e2c, o1, o2, on,
             ia, ib, ra, rb, p1, gsem):
    wid = lax.axis_index("s") * NC + lax.axis_index("c")

    def dots(buf):
        @plsc.parallel_loop(0, CHUNK, 1, unroll=4)
        def _(r):
            acc = ra[buf, r, pl.ds(0, LANES)] * rb[buf, r, pl.ds(0, LANES)]
            for b in range(1, DSUB):
                acc = acc + (ra[buf, r, pl.ds(LANES * b, LANES)]
                             * rb[buf, r, pl.ds(LANES * b, LANES)])
            p1[r, :] = acc

    def phase(idx_f_hbm, idx_t_hbm, tab_a, tab_b, out_hbm, nchunks):
        base = wid * nchunks
        pltpu.sync_copy(idx_f_hbm.at[wid], ia.at[pl.ds(0, nchunks)])
        pltpu.sync_copy(idx_t_hbm.at[wid], ib.at[pl.ds(0, nchunks)])

        def fetch(c, buf):
            pltpu.async_copy(tab_a.at[ia.at[c]], ra.at[buf], gsem.at[buf])
            pltpu.async_copy(tab_b.at[ib.at[c]], rb.at[buf], gsem.at[buf])

        def consume(c, buf):
            pltpu.make_async_copy(tab_a.at[ia.at[c]], ra.at[buf],
                                  gsem.at[buf]).wait()
            pltpu.make_async_copy(tab_b.at[ib.at[c]], rb.at[buf],
                                  gsem.at[buf]).wait()
            dots(buf)
            pltpu.sync_copy(p1, out_hbm.at[base + c])

        fetch(0, 0)

        def step(i, carry):
            for b in range(2):
                cc = i * 2 + b

                @pl.when(cc + 1 < nchunks)
                def _():
                    fetch(cc + 1, 1 - b)

                consume(cc, b)
            return carry

        lax.fori_loop(0, nchunks // 2, step, 0)

    phase(pf, pt, e1, e1, o1, POS_PER_W)
    phase(pf, pt, e2, e2c, o2, POS_PER_W)
    phase(nf, nt, e2, e2c, on, NEG_PER_W)


_sc_dots = pl.kernel(
    _sc_body,
    out_type=(
        jax.ShapeDtypeStruct((POS_BLKS, CHUNK, LANES), jnp.float32),
        jax.ShapeDtypeStruct((POS_BLKS, CHUNK, LANES), jnp.float32),
        jax.ShapeDtypeStruct((NEG_BLKS, CHUNK, LANES), jnp.float32),
    ),
    mesh=plsc.VectorSubcoreMesh(core_axis_name="c", subcore_axis_name="s"),
    scratch_types=(
        pltpu.VMEM((NEG_PER_W, CHUNK), jnp.int32),
        pltpu.VMEM((NEG_PER_W, CHUNK), jnp.int32),
        pltpu.VMEM((2, CHUNK, D), jnp.float32),
        pltpu.VMEM((2, CHUNK, D), jnp.float32),
        pltpu.VMEM((CHUNK, LANES), jnp.float32),
        pltpu.SemaphoreType.DMA((2,)),
    ),
)


def _log_sigmoid(x):
    return jnp.minimum(x, 0.0) - jnp.log(1.0 + jnp.exp(-jnp.abs(x)))


def _reduce_body(o1, o2, on, w, first_ref, second_ref):
    s1 = jnp.sum(o1[...], axis=-1)
    s2 = jnp.sum(o2[...], axis=-1)
    sn = jnp.sum(on[...], axis=-1)
    first = -jnp.sum(w[...] * _log_sigmoid(s1))
    pos_loss = jnp.sum(_log_sigmoid(s2))
    neg_loss = jnp.sum(_log_sigmoid(-sn))
    first_ref[0, 0] = first
    second_ref[0, 0] = -(pos_loss + neg_loss)


_reduce = pl.pallas_call(
    _reduce_body,
    out_shape=(
        jax.ShapeDtypeStruct((1, 1), jnp.float32),
        jax.ShapeDtypeStruct((1, 1), jnp.float32),
    ),
    out_specs=(
        pl.BlockSpec(memory_space=pltpu.SMEM),
        pl.BlockSpec(memory_space=pltpu.SMEM),
    ),
)


def kernel(pos, pos_w, neg, embed_1, embed_2, embed_2_context):
    pf = pos[:, 0].reshape(NW, POS_PER_W, CHUNK)
    pt = pos[:, 1].reshape(NW, POS_PER_W, CHUNK)
    nf = neg[:, 0].reshape(NW, NEG_PER_W, CHUNK)
    nt = neg[:, 1].reshape(NW, NEG_PER_W, CHUNK)

    o1, o2, on = _sc_dots(pf, pt, nf, nt, embed_1, embed_2, embed_2_context)

    first, second = _reduce(o1, o2, on, pos_w.reshape(POS_BLKS, CHUNK))
    return first[0, 0], second[0, 0]
